# baseline (speedup 1.0000x reference)
import jax
import jax.numpy as jnp
from jax import lax
from jax.experimental import pallas as pl
from jax.experimental.pallas import tpu as pltpu
from jax.experimental.pallas import tpu_sc as plsc

BATCH = 4096
SEQ = 200
EMBED = 64
NC = 2
NS = 16
NW = NC * NS
BLK = BATCH // NW
SHI = SEQ // 8


def _body(ids_hbm, table_hbm, out_hbm, ibuf, gbuf, tbuf,
          gsem0, gsem1, wsem0, wsem1):
    gsem = (gsem0, gsem1)
    wsem = (wsem0, wsem1)
    w = lax.axis_index("s") * NC + lax.axis_index("c")

    pltpu.sync_copy(ids_hbm.at[:, w], ibuf)

    def gather_start(s, p):
        pltpu.async_copy(table_hbm.at[ibuf.at[s // 8, s % 8]],
                         gbuf.at[p], gsem[p])

    def gather_wait(s, p):
        pltpu.make_async_copy(table_hbm.at[ibuf.at[s // 8, s % 8]],
                              gbuf.at[p], gsem[p]).wait()

    def write_start(s, p):
        pltpu.async_copy(tbuf.at[p], out_hbm.at[s, :, w], wsem[p])

    def write_wait(s, p):
        pltpu.make_async_copy(tbuf.at[p], out_hbm.at[s, :, w],
                              wsem[p]).wait()

    def transpose(p):
        biota = lax.iota(jnp.int32, 16)

        def erow(e, carry):
            evec = jnp.full((16,), 0, jnp.int32) + e
            for g in range(8):
                v = plsc.load_gather(gbuf.at[p], [biota + (16 * g), evec])
                tbuf[p, e // 8, e % 8, pl.ds(16 * g, 16)] = v
            return carry

        lax.fori_loop(0, EMBED, erow, 0)

    gather_start(0, 0)
    gather_start(1, 1)

    def step(t, carry):
        for p in (0, 1):
            s = 2 * t + p
            gather_wait(s, p)

            @pl.when(t >= 1)
            def _(s=s, p=p):
                write_wait(s - 2, p)

            transpose(p)

            @pl.when(t < SEQ // 2 - 1)
            def _(s=s, p=p):
                gather_start(s + 2, p)

            write_start(s, p)
        return carry

    lax.fori_loop(0, SEQ // 2, step, 0)
    write_wait(SEQ - 2, 0)
    write_wait(SEQ - 1, 1)


def kernel(input_ids, table):
    ids4 = (input_ids.T.reshape(SHI, 8, NW, BLK).transpose(0, 2, 1, 3))
    mesh = plsc.VectorSubcoreMesh(core_axis_name="c", subcore_axis_name="s")
    out5 = pl.kernel(
        _body,
        out_type=jax.ShapeDtypeStruct((SEQ, 8, NW, 8, BLK), jnp.float32),
        mesh=mesh,
        scratch_types=[
            pltpu.VMEM((SHI, 8, BLK), jnp.int32),
            pltpu.VMEM((2, BLK, EMBED), jnp.float32),
            pltpu.VMEM((2, 8, 8, BLK), jnp.float32),
            pltpu.SemaphoreType.DMA,
            pltpu.SemaphoreType.DMA,
            pltpu.SemaphoreType.DMA,
            pltpu.SemaphoreType.DMA,
        ],
        compiler_params=pltpu.CompilerParams(use_tc_tiling_on_sc=False,
                                             needs_layout_passes=False),
    )(ids4, table)
    return (out5.transpose(2, 4, 0, 1, 3)
            .reshape(BATCH, SEQ, EMBED))


# speedup vs baseline: 1.6681x; 1.6681x over previous
import jax
import jax.numpy as jnp
from jax import lax
from jax.experimental import pallas as pl
from jax.experimental.pallas import tpu as pltpu
from jax.experimental.pallas import tpu_sc as plsc

BATCH = 4096
SEQ = 200
EMBED = 64
NC = 2
NS = 16
NW = NC * NS
BLK = BATCH // NW
SHI = SEQ // 8
NQ = SEQ * NW // 2

H = 4
NGROUP = SEQ // H
NPAIR = NGROUP // 2


def _sc_body(ids_hbm, table_hbm, inter_hbm, ibuf, gbuf, *sems):
    gsem = sems[:2 * H]
    wsem = sems[2 * H:]
    w = lax.axis_index("s") * NC + lax.axis_index("c")
    q0 = w // 2
    colp = w % 2

    pltpu.sync_copy(ids_hbm.at[:, w], ibuf)

    def gather_start(s, b):
        pltpu.async_copy(table_hbm.at[ibuf.at[s // 8, s % 8]],
                         gbuf.at[b], gsem[b])

    def gather_wait(s, b):
        pltpu.make_async_copy(table_hbm.at[ibuf.at[s // 8, s % 8]],
                              gbuf.at[b], gsem[b]).wait()

    def _dst(s):
        return inter_hbm.at[s * 16 + q0, :, pl.ds(64 * colp, 64)]

    def write_start(s, b):
        pltpu.async_copy(gbuf.at[b], _dst(s), wsem[b])

    def write_wait(s, b):
        pltpu.make_async_copy(gbuf.at[b], _dst(s), wsem[b]).wait()

    for b in range(H):
        gather_start(b, b)

    def body(u, carry):
        for p in (0, 1):
            g = 2 * u + p
            c0 = g * H
            for b in range(H):
                gather_wait(c0 + b, p * H + b)
            for b in range(H):
                write_start(c0 + b, p * H + b)

            @pl.when(g >= 1)
            def _(c0=c0, p=p):
                for b in range(H):
                    write_wait(c0 - H + b, (1 - p) * H + b)

            @pl.when(g < NGROUP - 1)
            def _(c0=c0, p=p):
                for b in range(H):
                    gather_start(c0 + H + b, (1 - p) * H + b)
        return carry

    lax.fori_loop(0, NPAIR, body, 0)
    for b in range(H):
        write_wait(SEQ - H + b, H + b)


QB = 8


def _tc_body(in_ref, out_ref):
    for i in range(QB):
        t = in_ref[i].T
        t4 = t.reshape(2, 8, 8, BLK)
        out_ref[0, :, 2 * i, :, :] = t4[0]
        out_ref[0, :, 2 * i + 1, :, :] = t4[1]


def kernel(input_ids, table):
    ids4 = input_ids.T.reshape(SHI, 8, NW, BLK).transpose(0, 2, 1, 3)
    mesh = plsc.VectorSubcoreMesh(core_axis_name="c", subcore_axis_name="s")
    inter = pl.kernel(
        _sc_body,
        out_type=jax.ShapeDtypeStruct((NQ, BLK, BLK), jnp.float32),
        mesh=mesh,
        scratch_types=(
            [pltpu.VMEM((SHI, 8, BLK), jnp.int32),
             pltpu.VMEM((2 * H, BLK, EMBED), jnp.float32)]
            + [pltpu.SemaphoreType.DMA] * (4 * H)
        ),
        compiler_params=pltpu.CompilerParams(use_tc_tiling_on_sc=False,
                                             needs_layout_passes=False),
    )(ids4, table)

    out5 = pl.pallas_call(
        _tc_body,
        out_shape=jax.ShapeDtypeStruct((SEQ, 8, NW, 8, BLK), jnp.float32),
        grid=(NQ // QB,),
        in_specs=[pl.BlockSpec((QB, BLK, BLK), lambda t: (t, 0, 0))],
        out_specs=pl.BlockSpec((1, 8, 2 * QB, 8, BLK),
                               lambda t: (t // 2, 0, t % 2, 0, 0)),
    )(inter)

    return out5.transpose(2, 4, 0, 1, 3).reshape(BATCH, SEQ, EMBED)


# speedup vs baseline: 1.8287x; 1.0963x over previous
import jax
import jax.numpy as jnp
from jax import lax
from jax.experimental import pallas as pl
from jax.experimental.pallas import tpu as pltpu
from jax.experimental.pallas import tpu_sc as plsc

BATCH = 4096
SEQ = 200
EMBED = 64
NC = 2
NS = 16
NW = NC * NS
BLK = BATCH // NW
SHI = SEQ // 8
NQ = SEQ * NW // 2

H = 4
NGROUP = SEQ // H
NPAIR = NGROUP // 2


def _sc_body(ids_hbm, table_hbm, inter_hbm, ibuf, gbuf, *sems):
    gsem = sems[:2 * H]
    wsem = sems[2 * H:]
    w = lax.axis_index("s") * NC + lax.axis_index("c")
    q0 = w // 2
    colp = w % 2

    pltpu.sync_copy(ids_hbm.at[:, w], ibuf)

    def gather_start(s, b):
        pltpu.async_copy(table_hbm.at[ibuf.at[s // 8, s % 8]],
                         gbuf.at[b], gsem[b])

    def gather_wait(s, b):
        pltpu.make_async_copy(table_hbm.at[ibuf.at[s // 8, s % 8]],
                              gbuf.at[b], gsem[b]).wait()

    def _dst(s):
        return inter_hbm.at[s * 16 + q0, :, pl.ds(64 * colp, 64)]

    def write_start(s, b):
        pltpu.async_copy(gbuf.at[b], _dst(s), wsem[b])

    def write_wait(s, b):
        pltpu.make_async_copy(gbuf.at[b], _dst(s), wsem[b]).wait()

    for b in range(H):
        gather_start(b, b)

    def body(u, carry):
        for p in (0, 1):
            g = 2 * u + p
            c0 = g * H
            for b in range(H):
                gather_wait(c0 + b, p * H + b)
            for b in range(H):
                write_start(c0 + b, p * H + b)

            @pl.when(g >= 1)
            def _(c0=c0, p=p):
                for b in range(H):
                    write_wait(c0 - H + b, (1 - p) * H + b)

            @pl.when(g < NGROUP - 1)
            def _(c0=c0, p=p):
                for b in range(H):
                    gather_start(c0 + H + b, (1 - p) * H + b)
        return carry

    lax.fori_loop(0, NPAIR, body, 0)
    for b in range(H):
        write_wait(SEQ - H + b, H + b)


QB = 16


def _tc_body(in_ref, out_ref):
    for i in range(QB):
        t = in_ref[i].T
        t4 = t.reshape(2, 8, 8, BLK)
        out_ref[0, :, 2 * i, :, :] = t4[0]
        out_ref[0, :, 2 * i + 1, :, :] = t4[1]


def kernel(input_ids, table):
    ids4 = input_ids.T.reshape(SHI, 8, NW, BLK).transpose(0, 2, 1, 3)
    mesh = plsc.VectorSubcoreMesh(core_axis_name="c", subcore_axis_name="s")
    inter = pl.kernel(
        _sc_body,
        out_type=jax.ShapeDtypeStruct((NQ, BLK, BLK), jnp.float32),
        mesh=mesh,
        scratch_types=(
            [pltpu.VMEM((SHI, 8, BLK), jnp.int32),
             pltpu.VMEM((2 * H, BLK, EMBED), jnp.float32)]
            + [pltpu.SemaphoreType.DMA] * (4 * H)
        ),
        compiler_params=pltpu.CompilerParams(use_tc_tiling_on_sc=False,
                                             needs_layout_passes=False),
    )(ids4, table)

    out5 = pl.pallas_call(
        _tc_body,
        out_shape=jax.ShapeDtypeStruct((SEQ, 8, NW, 8, BLK), jnp.float32),
        grid=(NQ // QB,),
        in_specs=[pl.BlockSpec((QB, BLK, BLK), lambda t: (t, 0, 0))],
        out_specs=pl.BlockSpec((1, 8, 2 * QB, 8, BLK),
                               lambda t: (t, 0, 0, 0, 0)),
    )(inter)

    return out5.transpose(2, 4, 0, 1, 3).reshape(BATCH, SEQ, EMBED)
